# baseline (speedup 1.0000x reference)
import functools

import numpy as np
import jax
import jax.numpy as jnp
from jax import lax
from jax.experimental import pallas as pl
from jax.experimental.pallas import tpu as pltpu
from jax.experimental.pallas import tpu_sc as plsc

_BATCH = 16384
_NUM_FEATURE = 26
_EMB = 16
_BINS = 50
_TOTAL = 100
_INV = 3

_N = _BATCH * _NUM_FEATURE
_IPS = 128
_NROWS = _N // _IPS
_K = 8

_j = np.arange(_BINS)[:, None]
_c = np.arange(3 * _TOTAL)[None, :]
_MHB = (np.abs(_c - (_j + _TOTAL)) <= _INV).astype(np.float32)


def _table_body(mhb_ref, w_ref, e_ref):
    e_ref[...] = lax.dot_general(
        mhb_ref[...], w_ref[...], (((1,), (1,)), ((), ())),
        preferred_element_type=jnp.float32)


def kernel(x, W):
    table = pl.pallas_call(
        _table_body,
        out_shape=jax.ShapeDtypeStruct((_BINS, _EMB), jnp.float32),
    )(jnp.asarray(_MHB), W)

    idx = x.astype(jnp.int32).reshape(_NROWS, _IPS)

    info = plsc.get_sparse_core_info()
    nc, ns = info.num_cores, info.num_subcores
    nw = nc * ns
    rows_per_w = _NROWS // nw
    n_groups = rows_per_w // _K

    mesh = plsc.VectorSubcoreMesh(core_axis_name="c", subcore_axis_name="s")

    @functools.partial(
        pl.kernel,
        out_type=jax.ShapeDtypeStruct((_NROWS, _IPS, _EMB), jnp.float32),
        mesh=mesh,
        scratch_types=[
            pltpu.VMEM((rows_per_w, _IPS), jnp.int32),
            pltpu.VMEM((_K, _IPS, _EMB), jnp.float32),
            pltpu.SemaphoreType.DMA,
        ],
        compiler_params=pltpu.CompilerParams(use_tc_tiling_on_sc=False),
    )
    def _gather(tab_hbm, idx_hbm, out_hbm, idx_v, rows_v, sem):
        wid = lax.axis_index("s") * nc + lax.axis_index("c")
        base = wid * rows_per_w
        pltpu.sync_copy(idx_hbm.at[pl.ds(base, rows_per_w)], idx_v)

        def group(g, carry):
            handles = []
            for b in range(_K):
                handles.append(pltpu.async_copy(
                    tab_hbm.at[idx_v.at[g * _K + b]], rows_v.at[b], sem))
            for h in handles:
                h.wait()
            pltpu.sync_copy(rows_v, out_hbm.at[pl.ds(base + g * _K, _K)])
            return carry

        lax.fori_loop(0, n_groups, group, 0)

    out = _gather(table, idx)
    return out.reshape(_BATCH, _NUM_FEATURE * _EMB)
